# baseline (speedup 1.0000x reference)
import functools

import jax
import jax.numpy as jnp
from jax.experimental import pallas as pl
from jax.experimental.pallas import tpu as pltpu
from jax.experimental.pallas import tpu_sc as plsc

_W = 128
_PACK_BLOCK = 8192
_MM_BLOCK = 512


def _pack_body(a_ref, o_ref):
    a = a_ref[...]
    a3 = a.reshape(a.shape[0] // 4, 4, 32)
    for j in range(4):
        o_ref[:, 32 * j:32 * (j + 1)] = a3[:, j, :]


@jax.jit
def _tc_pack(table):
    n, r = table.shape
    return pl.pallas_call(
        _pack_body,
        grid=(n // _PACK_BLOCK,),
        in_specs=[pl.BlockSpec((_PACK_BLOCK, r), lambda i: (i, 0))],
        out_specs=pl.BlockSpec((_PACK_BLOCK // 4, 4 * r), lambda i: (i, 0)),
        out_shape=jax.ShapeDtypeStruct((n // 4, 4 * r), table.dtype),
    )(table)


@functools.partial(jax.jit, static_argnums=(2, 3))
def _sc_gather(packed_table, idx_2d, num_rows, rank):
    n_steps = idx_2d.shape[0]
    mesh = plsc.VectorSubcoreMesh(core_axis_name="core", subcore_axis_name="subcore")

    @functools.partial(
        pl.kernel,
        out_type=jax.ShapeDtypeStruct((n_steps * _W, rank), jnp.float32),
        mesh=mesh,
        compiler_params=pltpu.CompilerParams(use_tc_tiling_on_sc=False),
    )
    def gather_kernel(tbl_hbm, idx_hbm, out_hbm):
        def body(i_vmem, o_vmem):
            pltpu.sync_copy(tbl_hbm.at[i_vmem.at[0]], o_vmem)

        pltpu.emit_pipeline(
            body,
            grid=(n_steps,),
            in_specs=[pl.BlockSpec((1, _W), lambda i: (i, 0))],
            out_specs=[pl.BlockSpec((_W, rank), lambda i: (i, 0))],
            core_axis_name=("core", "subcore"),
            dimension_semantics=(pltpu.PARALLEL,),
        )(idx_hbm, out_hbm)

    return gather_kernel(packed_table, idx_2d)


def _mm_body(a_ref, b_ref, o_ref):
    a = a_ref[...]
    b = b_ref[...]
    for j in range(4):
        o_ref[:, 128 * j:128 * (j + 1)] = jnp.dot(
            a[:, 32 * j:32 * (j + 1)], b, preferred_element_type=jnp.float32)


@jax.jit
def _tc_project(a_packed, lf):
    n = a_packed.shape[0]
    k, m = lf.shape
    return pl.pallas_call(
        _mm_body,
        grid=(n // _MM_BLOCK,),
        in_specs=[
            pl.BlockSpec((_MM_BLOCK, 128), lambda i: (i, 0)),
            pl.BlockSpec((k, m), lambda i: (0, 0)),
        ],
        out_specs=pl.BlockSpec((_MM_BLOCK, 4 * m), lambda i: (i, 0)),
        out_shape=jax.ShapeDtypeStruct((n, 4 * m), jnp.float32),
    )(a_packed, lf)


def kernel(x, first_factor, last_factor):
    num_rows, rank = first_factor.shape
    emb_dim = last_factor.shape[1]
    num_idx = x.size

    idx_2d = x.reshape(-1).astype(jnp.int32).reshape(num_idx // _W, _W)
    packed_table = _tc_pack(first_factor).reshape(num_rows, rank)
    gathered = _sc_gather(packed_table, idx_2d, num_rows, rank)
    gathered_packed = gathered.reshape(num_idx * rank // 128, 128)
    out = _tc_project(gathered_packed, last_factor)
    return out.reshape(tuple(x.shape) + (emb_dim,))
